# baseline (speedup 1.0000x reference)
import jax
import jax.numpy as jnp
from jax.experimental import pallas as pl
from jax.experimental.pallas import tpu as pltpu

BLOCK_NUM = 256


def _sgc_kernel(rm_ref, adj_hbm, out_ref, adj_vmem, sems):
    nb = adj_vmem.shape[0]
    for k in range(nb):
        pltpu.make_async_copy(adj_hbm.at[k], adj_vmem.at[k], sems.at[k]).start()
    for k in range(nb):
        pltpu.make_async_copy(adj_hbm.at[k], adj_vmem.at[k], sems.at[k]).wait()
        adj = adj_vmem[k]
        rm = rm_ref[k]
        a2 = jnp.dot(adj, adj, preferred_element_type=jnp.float32).astype(
            jnp.bfloat16)
        t = jnp.dot(a2, rm, preferred_element_type=jnp.float32)
        full = jnp.dot(a2, t, preferred_element_type=jnp.float32)
        diag = jnp.sum((a2 * a2.T).astype(jnp.float32), axis=1, keepdims=True)
        out_ref[k] = full - diag * rm


def kernel(regional_means, adj):
    b, n, d = regional_means.shape
    return pl.pallas_call(
        _sgc_kernel,
        in_specs=[
            pl.BlockSpec(memory_space=pltpu.MemorySpace.VMEM),
            pl.BlockSpec(memory_space=pltpu.MemorySpace.HBM),
        ],
        out_specs=pl.BlockSpec(memory_space=pltpu.MemorySpace.VMEM),
        out_shape=jax.ShapeDtypeStruct((b, n, d), jnp.float32),
        scratch_shapes=[
            pltpu.VMEM((b, n, n), jnp.float32),
            pltpu.SemaphoreType.DMA((b,)),
        ],
    )(regional_means, adj)


# speedup vs baseline: 1.2059x; 1.2059x over previous
import jax
import jax.numpy as jnp
from jax.experimental import pallas as pl
from jax.experimental.pallas import tpu as pltpu

BLOCK_NUM = 256
GROUPS = 4


def _sgc_kernel(rm_ref, adj_hbm, out_ref, adj_vmem, sems):
    nb = adj_vmem.shape[0]
    gb = nb // GROUPS
    for g in range(GROUPS):
        pltpu.make_async_copy(
            adj_hbm.at[pl.ds(g * gb, gb)],
            adj_vmem.at[pl.ds(g * gb, gb)],
            sems.at[g],
        ).start()
    for g in range(GROUPS):
        pltpu.make_async_copy(
            adj_hbm.at[pl.ds(g * gb, gb)],
            adj_vmem.at[pl.ds(g * gb, gb)],
            sems.at[g],
        ).wait()
        for k in range(g * gb, (g + 1) * gb):
            adj = adj_vmem[k]
            rm = rm_ref[k]
            a2 = jnp.dot(adj, adj, preferred_element_type=jnp.float32).astype(
                jnp.bfloat16)
            t = jnp.dot(a2, rm, preferred_element_type=jnp.float32)
            full = jnp.dot(a2, t, preferred_element_type=jnp.float32)
            diag = jnp.sum((a2 * a2.T).astype(jnp.float32), axis=1,
                           keepdims=True)
            out_ref[k] = full - diag * rm


def kernel(regional_means, adj):
    b, n, d = regional_means.shape
    return pl.pallas_call(
        _sgc_kernel,
        in_specs=[
            pl.BlockSpec(memory_space=pltpu.MemorySpace.VMEM),
            pl.BlockSpec(memory_space=pltpu.MemorySpace.HBM),
        ],
        out_specs=pl.BlockSpec(memory_space=pltpu.MemorySpace.VMEM),
        out_shape=jax.ShapeDtypeStruct((b, n, d), jnp.float32),
        scratch_shapes=[
            pltpu.VMEM((b, n, n), jnp.float32),
            pltpu.SemaphoreType.DMA((GROUPS,)),
        ],
    )(regional_means, adj)


# speedup vs baseline: 1.3735x; 1.1390x over previous
import jax
import jax.numpy as jnp
from jax.experimental import pallas as pl

BLOCK_NUM = 256

NSPLIT = 4
BB = 8
SB = BB // NSPLIT


def _sgc_kernel(rm_ref, *refs):
    adj_refs = refs[:NSPLIT]
    out_ref = refs[NSPLIT]
    for j in range(NSPLIT):
        for k in range(SB):
            adj = adj_refs[j][k]
            rm = rm_ref[j * SB + k]
            a2 = jnp.dot(adj, adj, preferred_element_type=jnp.float32).astype(
                jnp.bfloat16)
            t = jnp.dot(a2, rm, preferred_element_type=jnp.float32)
            full = jnp.dot(a2, t, preferred_element_type=jnp.float32)
            diag = jnp.sum((a2 * a2.T).astype(jnp.float32), axis=1,
                           keepdims=True)
            out_ref[j * SB + k] = full - diag * rm


def _adj_index_map(j):
    return lambda i: (i * NSPLIT + j, 0, 0)


def kernel(regional_means, adj):
    b, n, d = regional_means.shape
    in_specs = [pl.BlockSpec((BB, n, d), lambda i: (i, 0, 0))]
    in_specs += [
        pl.BlockSpec((SB, n, n), _adj_index_map(j)) for j in range(NSPLIT)
    ]
    return pl.pallas_call(
        _sgc_kernel,
        grid=(b // BB,),
        in_specs=in_specs,
        out_specs=pl.BlockSpec((BB, n, d), lambda i: (i, 0, 0)),
        out_shape=jax.ShapeDtypeStruct((b, n, d), jnp.float32),
    )(regional_means, *([adj] * NSPLIT))


# speedup vs baseline: 3.1457x; 2.2902x over previous
import jax
import jax.numpy as jnp
from jax.experimental import pallas as pl

BLOCK_NUM = 256
BB = 8


def _sgc_kernel(rmt_ref, adj_ref, out_ref):
    for k in range(BB):
        a = adj_ref[k]
        rmt = rmt_ref[k]
        a2 = jnp.dot(a, a, preferred_element_type=jnp.float32)
        b = a2.T
        u = jnp.dot(rmt, b, preferred_element_type=jnp.float32)
        full_t = jnp.dot(u, b, preferred_element_type=jnp.float32)
        diag = jnp.sum(a2 * b, axis=0, keepdims=True)
        out_ref[k] = full_t - rmt * diag


def kernel(regional_means, adj):
    bsz, n, d = regional_means.shape
    rm_t = jnp.transpose(regional_means, (0, 2, 1))
    out_t = pl.pallas_call(
        _sgc_kernel,
        grid=(bsz // BB,),
        in_specs=[
            pl.BlockSpec((BB, d, n), lambda i: (i, 0, 0)),
            pl.BlockSpec((BB, n, n), lambda i: (i, 0, 0)),
        ],
        out_specs=pl.BlockSpec((BB, d, n), lambda i: (i, 0, 0)),
        out_shape=jax.ShapeDtypeStruct((bsz, d, n), jnp.float32),
    )(rm_t, adj)
    return jnp.transpose(out_t, (0, 2, 1))


# speedup vs baseline: 3.2136x; 1.0216x over previous
import jax
import jax.numpy as jnp
from jax.experimental import pallas as pl
from jax.experimental.pallas import tpu as pltpu

BLOCK_NUM = 256
BB = 8


def _sgc_kernel(rmt_ref, adj_ref, out_ref):
    for k in range(BB):
        a = adj_ref[k]
        rmt = rmt_ref[k]
        b = jax.lax.dot_general(
            a, a, (((0,), (1,)), ((), ())),
            preferred_element_type=jnp.float32)
        u = jnp.dot(rmt, b, preferred_element_type=jnp.float32)
        full_t = jnp.dot(u, b, preferred_element_type=jnp.float32)
        diag = jnp.sum(b * b.T, axis=0, keepdims=True)
        out_ref[k] = full_t - rmt * diag


def kernel(regional_means, adj):
    bsz, n, d = regional_means.shape
    rm_t = jnp.transpose(regional_means, (0, 2, 1))
    adj = pltpu.with_memory_space_constraint(adj, pltpu.MemorySpace.HBM)
    out_t = pl.pallas_call(
        _sgc_kernel,
        grid=(bsz // BB,),
        in_specs=[
            pl.BlockSpec((BB, d, n), lambda i: (i, 0, 0)),
            pl.BlockSpec((BB, n, n), lambda i: (i, 0, 0)),
        ],
        out_specs=pl.BlockSpec((BB, d, n), lambda i: (i, 0, 0)),
        out_shape=jax.ShapeDtypeStruct((bsz, d, n), jnp.float32),
    )(rm_t, adj)
    return jnp.transpose(out_t, (0, 2, 1))


# speedup vs baseline: 3.9686x; 1.2349x over previous
import jax
import jax.numpy as jnp
from jax.experimental import pallas as pl
from jax.experimental.pallas import tpu as pltpu

BLOCK_NUM = 256
BB = 8


def _sgc_kernel(rmt_ref, adj_ref, out_ref):
    bs = [
        jax.lax.dot_general(
            adj_ref[k], adj_ref[k], (((0,), (1,)), ((), ())),
            preferred_element_type=jnp.float32)
        for k in range(BB)
    ]
    us = [
        jnp.dot(rmt_ref[k], bs[k], preferred_element_type=jnp.float32)
        for k in range(BB)
    ]
    fulls = [
        jnp.dot(us[k], bs[k], preferred_element_type=jnp.float32)
        for k in range(BB)
    ]
    diags = [
        jnp.sum(bs[k] * bs[k].T, axis=0, keepdims=True) for k in range(BB)
    ]
    for k in range(BB):
        out_ref[k] = fulls[k] - rmt_ref[k] * diags[k]


def kernel(regional_means, adj):
    bsz, n, d = regional_means.shape
    rm_t = jnp.transpose(regional_means, (0, 2, 1))
    adj = pltpu.with_memory_space_constraint(adj, pltpu.MemorySpace.HBM)
    out_t = pl.pallas_call(
        _sgc_kernel,
        grid=(bsz // BB,),
        in_specs=[
            pl.BlockSpec((BB, d, n), lambda i: (i, 0, 0)),
            pl.BlockSpec((BB, n, n), lambda i: (i, 0, 0)),
        ],
        out_specs=pl.BlockSpec((BB, d, n), lambda i: (i, 0, 0)),
        out_shape=jax.ShapeDtypeStruct((bsz, d, n), jnp.float32),
    )(rm_t, adj)
    return jnp.transpose(out_t, (0, 2, 1))
